# baseline (speedup 1.0000x reference)
import jax
import jax.numpy as jnp
from jax.experimental import pallas as pl

_ALPHA = 0.25
_B = 16384
_T = 200
_BK = 4096


def _body(yp_ref, y_ref, st_ref, out_ref):
    i = pl.program_id(0)
    p = yp_ref[...]
    yb = y_ref[pl.ds(i * _BK, _BK)].reshape(_BK, 1)
    wu = st_ref[pl.ds(i * _BK, _BK)].reshape(_BK, 1).astype(jnp.float32)

    t = jax.lax.broadcasted_iota(jnp.int32, p.shape, 1)
    l1m = jnp.log(1.0 - p)
    m_le = t <= yb
    m_eq = t == yb
    ones_t = jnp.ones((_T, 1), jnp.float32)

    def _rowsum(x):
        return jax.lax.dot_general(
            x, ones_t, (((1,), (0,)), ((), ())),
            preferred_element_type=jnp.float32)

    s_y = _rowsum(jnp.where(m_le, l1m, 0.0))
    l1m_y = _rowsum(jnp.where(m_eq, l1m, 0.0))
    p_y = _rowsum(jnp.where(m_eq, p, 0.0))

    s_ym1 = jnp.where(yb >= 1, s_y - l1m_y, 0.0)
    lz = wu * (jnp.log(p_y) + s_ym1)
    lu = wu * jnp.log(1.0 - jnp.exp(s_y))
    lc = (1.0 - wu) * s_y
    total = _ALPHA * (-jnp.sum(lz)) + (1.0 - _ALPHA) * (-(jnp.sum(lu) + jnp.sum(lc)))

    @pl.when(i == 0)
    def _init():
        out_ref[...] = jnp.zeros_like(out_ref)

    out_ref[...] += total


@jax.jit
def kernel(y_pred, y, status):
    grid = _B // _BK
    out = pl.pallas_call(
        _body,
        grid=(grid,),
        in_specs=[
            pl.BlockSpec((_BK, _T), lambda i: (i, 0)),
            pl.BlockSpec((_B,), lambda i: (0,)),
            pl.BlockSpec((_B,), lambda i: (0,)),
        ],
        out_specs=pl.BlockSpec((1, 1), lambda i: (0, 0)),
        out_shape=jax.ShapeDtypeStruct((1, 1), jnp.float32),
    )(y_pred, y.astype(jnp.int32), status.astype(jnp.int32))
    return out[0, 0]


# speedup vs baseline: 1.0206x; 1.0206x over previous
import jax
import jax.numpy as jnp
from jax.experimental import pallas as pl

_ALPHA = 0.25
_B = 16384
_T = 200
_BK = 4096


def _body(yp_ref, y_ref, st_ref, out_ref):
    i = pl.program_id(0)
    p = yp_ref[...]
    yb = y_ref[pl.ds(i * _BK, _BK)].reshape(_BK, 1)
    wu = st_ref[pl.ds(i * _BK, _BK)].reshape(_BK, 1).astype(jnp.float32)

    t = jax.lax.broadcasted_iota(jnp.int32, p.shape, 1)
    l1m = jnp.log(1.0 - p)
    m_le = t <= yb
    m_eq = t == yb
    ones_t = jnp.ones((_T, 1), jnp.float32)

    def _rowsum(x):
        return jax.lax.dot_general(
            x, ones_t, (((1,), (0,)), ((), ())),
            preferred_element_type=jnp.float32)

    s_y = _rowsum(jnp.where(m_le, l1m, 0.0))
    l1m_y = _rowsum(jnp.where(m_eq, l1m, 0.0))

    s_ym1 = jnp.where(yb >= 1, s_y - l1m_y, 0.0)
    lz = wu * (jnp.log(1.0 - jnp.exp(l1m_y)) + s_ym1)
    lu = wu * jnp.log(1.0 - jnp.exp(s_y))
    lc = (1.0 - wu) * s_y
    total = _ALPHA * (-jnp.sum(lz)) + (1.0 - _ALPHA) * (-(jnp.sum(lu) + jnp.sum(lc)))

    @pl.when(i == 0)
    def _init():
        out_ref[...] = jnp.zeros_like(out_ref)

    out_ref[...] += total


@jax.jit
def kernel(y_pred, y, status):
    grid = _B // _BK
    out = pl.pallas_call(
        _body,
        grid=(grid,),
        in_specs=[
            pl.BlockSpec((_BK, _T), lambda i: (i, 0)),
            pl.BlockSpec((_B,), lambda i: (0,)),
            pl.BlockSpec((_B,), lambda i: (0,)),
        ],
        out_specs=pl.BlockSpec((1, 1), lambda i: (0, 0)),
        out_shape=jax.ShapeDtypeStruct((1, 1), jnp.float32),
    )(y_pred, y.astype(jnp.int32), status.astype(jnp.int32))
    return out[0, 0]
